# baseline (speedup 1.0000x reference)
import functools

import jax
import jax.numpy as jnp
from jax import lax
from jax.experimental import pallas as pl
from jax.experimental.pallas import tpu as pltpu
from jax.experimental.pallas import tpu_sc as plsc


_CHUNK = 72
_NBUF = 5
_SEG = 8
_NCH_REAL = 139
_NCH_ARR = 144
_NTRASH = 16


def _make_sc_scatter(n_nodes, d):
  info = plsc.get_sparse_core_info()
  nc, ns = info.num_cores, info.num_subcores
  chunk = _CHUNK
  n_acc = n_nodes + _NTRASH
  n_seg = _NCH_ARR // _SEG
  rows_base = (n_nodes // (8 * ns)) * 8
  rows_tail = n_nodes - rows_base * ns

  mesh = plsc.VectorSubcoreMesh(core_axis_name="c", subcore_axis_name="s")

  @functools.partial(
      pl.kernel,
      out_type=jax.ShapeDtypeStruct((nc, n_nodes, d), jnp.float32),
      mesh=mesh,
      scratch_types=[
          [pltpu.VMEM((_SEG, chunk), jnp.int32)] * 2,
          [pltpu.VMEM((_SEG, chunk), jnp.int32)] * 2,
          [pltpu.VMEM((chunk, d), jnp.float32)] * _NBUF,
          pltpu.VMEM_SHARED((n_acc, d), jnp.float32),
          [pltpu.SemaphoreType.DMA] * _NBUF,
          [pltpu.SemaphoreType.DMA] * 2,
          [pltpu.SemaphoreType.DMA] * 2,
      ],
  )
  def sc_scatter(src_hbm, dst_hbm, x_hbm, out_hbm,
                 src_v, dst_v, rows, acc, gsem, issem, idsem):
    c = lax.axis_index("c")
    s = lax.axis_index("s")
    wid = s * nc + c
    my_src = src_hbm.at[wid]
    my_dst = dst_hbm.at[wid]

    row0 = s * rows_base
    pltpu.sync_copy(x_hbm.at[pl.ds(row0, rows_base)],
                    acc.at[pl.ds(row0, rows_base)])
    if rows_tail:
      @pl.when(s == ns - 1)
      def _():
        pltpu.sync_copy(x_hbm.at[pl.ds(ns * rows_base, rows_tail)],
                        acc.at[pl.ds(ns * rows_base, rows_tail)])
    plsc.subcore_barrier()

    pltpu.sync_copy(my_src.at[pl.ds(0, _SEG)], src_v[0])
    pltpu.sync_copy(my_dst.at[pl.ds(0, _SEG)], dst_v[0])
    for b in range(_NBUF):
      pltpu.async_copy(x_hbm.at[src_v[0].at[b]], rows[b], gsem[b])

    for j in range(_NCH_REAL):
      k, r = divmod(j, _SEG)
      b = j % _NBUF
      p = k % 2
      if r == 0 and k + 1 < n_seg:
        pn = (k + 1) % 2
        pltpu.async_copy(my_src.at[pl.ds((k + 1) * _SEG, _SEG)],
                         src_v[pn], issem[pn])
        pltpu.async_copy(my_dst.at[pl.ds((k + 1) * _SEG, _SEG)],
                         dst_v[pn], idsem[pn])
      jj = j + _NBUF
      if jj < _NCH_REAL and jj % _SEG < _NBUF:
        kk = jj // _SEG
        if jj % _SEG == 0:
          pp = kk % 2
          pltpu.make_async_copy(my_src.at[pl.ds(kk * _SEG, _SEG)],
                                src_v[pp], issem[pp]).wait()
          pltpu.make_async_copy(my_dst.at[pl.ds(kk * _SEG, _SEG)],
                                dst_v[pp], idsem[pp]).wait()
      pltpu.make_async_copy(x_hbm.at[src_v[p].at[r]], rows[b],
                            gsem[b]).wait()
      pltpu.sync_copy(rows[b], acc.at[dst_v[p].at[r]], add=True)
      if jj < _NCH_REAL:
        kk, rr = divmod(jj, _SEG)
        pltpu.async_copy(x_hbm.at[src_v[kk % 2].at[rr]], rows[b], gsem[b])
    plsc.subcore_barrier()

    pltpu.sync_copy(acc.at[pl.ds(row0, rows_base)],
                    out_hbm.at[c].at[pl.ds(row0, rows_base)])
    if rows_tail:
      @pl.when(s == ns - 1)
      def _():
        pltpu.sync_copy(acc.at[pl.ds(ns * rows_base, rows_tail)],
                        out_hbm.at[c].at[pl.ds(ns * rows_base, rows_tail)])

  return sc_scatter


_NB = 5
_BR = 2000


def _mlp_grid_body(eps_ref, x_ref, p_ref, w1_ref, b1_ref, g1_ref, be1_ref,
                   w2_ref, b2_ref, g2_ref, be2_ref, out_ref,
                   h1_s, h2_s, st1, st2):
  ph = pl.program_id(0)
  i = pl.program_id(1)
  n = jnp.float32(_NB * _BR)

  @pl.when(ph == 0)
  def _():
    @pl.when(i == 0)
    def _():
      st1[...] = jnp.zeros_like(st1)
    eps = eps_ref[0, 0]
    comb = (eps - 1.0) * x_ref[...] + p_ref[0] + p_ref[1]
    h = jnp.dot(comb, w1_ref[...], preferred_element_type=jnp.float32)
    h = h + b1_ref[...]
    h1_s[pl.ds(i * _BR, _BR), :] = h
    st1[0:1, :] += jnp.sum(h, axis=0, keepdims=True)
    st1[1:2, :] += jnp.sum(h * h, axis=0, keepdims=True)

  @pl.when(ph == 1)
  def _():
    @pl.when(i == 0)
    def _():
      st2[...] = jnp.zeros_like(st2)
    mean = st1[0:1, :] / n
    var = st1[1:2, :] / n - mean * mean
    h = h1_s[pl.ds(i * _BR, _BR), :]
    h = (h - mean) * lax.rsqrt(var + 1e-5) * g1_ref[...] + be1_ref[...]
    h = jnp.maximum(h, 0.0)
    h2 = jnp.dot(h, w2_ref[...], preferred_element_type=jnp.float32)
    h2 = h2 + b2_ref[...]
    h2_s[pl.ds(i * _BR, _BR), :] = h2
    st2[0:1, :] += jnp.sum(h2, axis=0, keepdims=True)
    st2[1:2, :] += jnp.sum(h2 * h2, axis=0, keepdims=True)

  @pl.when(ph == 2)
  def _():
    mean = st2[0:1, :] / n
    var = st2[1:2, :] / n - mean * mean
    h2 = h2_s[pl.ds(i * _BR, _BR), :]
    out_ref[...] = (h2 - mean) * lax.rsqrt(var + 1e-5) * g2_ref[...] \
        + be2_ref[...]


@jax.jit
def kernel(x, edge_index, epsilon, W1, b1, g1, be1, W2, b2, g2, be2):
  n_nodes, d = x.shape
  n_edges = edge_index.shape[1]
  nw = 32
  per_tile = n_edges // nw
  pad = _NCH_REAL * _CHUNK - per_tile
  filler = _NCH_ARR * _CHUNK - per_tile - pad

  sc_scatter = _make_sc_scatter(n_nodes, d)
  pad_ar = jnp.arange(pad, dtype=jnp.int32)
  src3 = jnp.concatenate([
      edge_index[0].reshape(nw, per_tile),
      jnp.broadcast_to(pad_ar % 128, (nw, pad)),
      jnp.zeros((nw, filler), jnp.int32),
  ], axis=1).reshape(nw, _NCH_ARR, _CHUNK)
  dst3 = jnp.concatenate([
      edge_index[1].reshape(nw, per_tile),
      jnp.broadcast_to(n_nodes + pad_ar % _NTRASH, (nw, pad)),
      jnp.zeros((nw, filler), jnp.int32),
  ], axis=1).reshape(nw, _NCH_ARR, _CHUNK)
  partials = sc_scatter(src3, dst3, x)

  d_hid = W1.shape[1]
  fix = lambda *blk: pl.BlockSpec(blk, lambda ph, i: (0,) * len(blk))
  mlp = pl.pallas_call(
      _mlp_grid_body,
      grid=(3, _NB),
      out_shape=jax.ShapeDtypeStruct((n_nodes, d), jnp.float32),
      in_specs=[
          pl.BlockSpec((1, 1), lambda ph, i: (0, 0),
                       memory_space=pltpu.SMEM),
          pl.BlockSpec((_BR, d),
                       lambda ph, i: (jnp.where(ph == 0, i, 0), 0)),
          pl.BlockSpec((2, _BR, d),
                       lambda ph, i: (0, jnp.where(ph == 0, i, 0), 0)),
          fix(d, d_hid),
          fix(1, d_hid), fix(1, d_hid), fix(1, d_hid),
          fix(d_hid, d),
          fix(1, d), fix(1, d), fix(1, d),
      ],
      out_specs=pl.BlockSpec((_BR, d),
                             lambda ph, i: (jnp.where(ph == 2, i, 0), 0)),
      scratch_shapes=[
          pltpu.VMEM((n_nodes, d_hid), jnp.float32),
          pltpu.VMEM((n_nodes, d), jnp.float32),
          pltpu.VMEM((2, d_hid), jnp.float32),
          pltpu.VMEM((2, d), jnp.float32),
      ],
  )
  return mlp(
      jnp.reshape(epsilon.astype(jnp.float32), (1, 1)),
      x, partials, W1,
      jnp.reshape(b1, (1, d_hid)), jnp.reshape(g1, (1, d_hid)),
      jnp.reshape(be1, (1, d_hid)),
      W2, jnp.reshape(b2, (1, d)), jnp.reshape(g2, (1, d)),
      jnp.reshape(be2, (1, d)))


# speedup vs baseline: 1.0154x; 1.0154x over previous
import functools

import jax
import jax.numpy as jnp
from jax import lax
from jax.experimental import pallas as pl
from jax.experimental.pallas import tpu as pltpu
from jax.experimental.pallas import tpu_sc as plsc


_CHUNK = 88
_NBUF = 4
_SEG = 8
_NCH_REAL = 114
_NCH_ARR = 120
_NTRASH = 16


def _make_sc_scatter(n_nodes, d):
  info = plsc.get_sparse_core_info()
  nc, ns = info.num_cores, info.num_subcores
  chunk = _CHUNK
  n_acc = n_nodes + _NTRASH
  n_seg = _NCH_ARR // _SEG
  rows_base = (n_nodes // (8 * ns)) * 8
  rows_tail = n_nodes - rows_base * ns

  mesh = plsc.VectorSubcoreMesh(core_axis_name="c", subcore_axis_name="s")

  @functools.partial(
      pl.kernel,
      out_type=jax.ShapeDtypeStruct((nc, n_nodes, d), jnp.float32),
      mesh=mesh,
      scratch_types=[
          [pltpu.VMEM((_SEG, chunk), jnp.int32)] * 2,
          [pltpu.VMEM((_SEG, chunk), jnp.int32)] * 2,
          [pltpu.VMEM((chunk, d), jnp.float32)] * _NBUF,
          pltpu.VMEM_SHARED((n_acc, d), jnp.float32),
          [pltpu.SemaphoreType.DMA] * _NBUF,
          [pltpu.SemaphoreType.DMA] * 2,
          [pltpu.SemaphoreType.DMA] * 2,
      ],
  )
  def sc_scatter(src_hbm, dst_hbm, x_hbm, out_hbm,
                 src_v, dst_v, rows, acc, gsem, issem, idsem):
    c = lax.axis_index("c")
    s = lax.axis_index("s")
    wid = s * nc + c
    my_src = src_hbm.at[wid]
    my_dst = dst_hbm.at[wid]

    row0 = s * rows_base
    pltpu.sync_copy(x_hbm.at[pl.ds(row0, rows_base)],
                    acc.at[pl.ds(row0, rows_base)])
    if rows_tail:
      @pl.when(s == ns - 1)
      def _():
        pltpu.sync_copy(x_hbm.at[pl.ds(ns * rows_base, rows_tail)],
                        acc.at[pl.ds(ns * rows_base, rows_tail)])
    plsc.subcore_barrier()

    pltpu.sync_copy(my_src.at[pl.ds(0, _SEG)], src_v[0])
    pltpu.sync_copy(my_dst.at[pl.ds(0, _SEG)], dst_v[0])
    for b in range(_NBUF):
      pltpu.async_copy(x_hbm.at[src_v[0].at[b]], rows[b], gsem[b])

    for j in range(_NCH_REAL):
      k, r = divmod(j, _SEG)
      b = j % _NBUF
      p = k % 2
      if r == 0 and k + 1 < n_seg:
        pn = (k + 1) % 2
        pltpu.async_copy(my_src.at[pl.ds((k + 1) * _SEG, _SEG)],
                         src_v[pn], issem[pn])
        pltpu.async_copy(my_dst.at[pl.ds((k + 1) * _SEG, _SEG)],
                         dst_v[pn], idsem[pn])
      jj = j + _NBUF
      if jj < _NCH_REAL and jj % _SEG < _NBUF:
        kk = jj // _SEG
        if jj % _SEG == 0:
          pp = kk % 2
          pltpu.make_async_copy(my_src.at[pl.ds(kk * _SEG, _SEG)],
                                src_v[pp], issem[pp]).wait()
          pltpu.make_async_copy(my_dst.at[pl.ds(kk * _SEG, _SEG)],
                                dst_v[pp], idsem[pp]).wait()
      pltpu.make_async_copy(x_hbm.at[src_v[p].at[r]], rows[b],
                            gsem[b]).wait()
      pltpu.sync_copy(rows[b], acc.at[dst_v[p].at[r]], add=True)
      if jj < _NCH_REAL:
        kk, rr = divmod(jj, _SEG)
        pltpu.async_copy(x_hbm.at[src_v[kk % 2].at[rr]], rows[b], gsem[b])
    plsc.subcore_barrier()

    pltpu.sync_copy(acc.at[pl.ds(row0, rows_base)],
                    out_hbm.at[c].at[pl.ds(row0, rows_base)])
    if rows_tail:
      @pl.when(s == ns - 1)
      def _():
        pltpu.sync_copy(acc.at[pl.ds(ns * rows_base, rows_tail)],
                        out_hbm.at[c].at[pl.ds(ns * rows_base, rows_tail)])

  return sc_scatter


_NB = 5
_BR = 2000


def _mlp_grid_body(eps_ref, x_ref, p_ref, w1_ref, b1_ref, g1_ref, be1_ref,
                   w2_ref, b2_ref, g2_ref, be2_ref, out_ref,
                   h1_s, h2_s, st1, st2):
  ph = pl.program_id(0)
  i = pl.program_id(1)
  n = jnp.float32(_NB * _BR)

  @pl.when(ph == 0)
  def _():
    @pl.when(i == 0)
    def _():
      st1[...] = jnp.zeros_like(st1)
    eps = eps_ref[0, 0]
    comb = (eps - 1.0) * x_ref[...] + p_ref[0] + p_ref[1]
    h = jnp.dot(comb, w1_ref[...], preferred_element_type=jnp.float32)
    h = h + b1_ref[...]
    h1_s[pl.ds(i * _BR, _BR), :] = h
    st1[0:1, :] += jnp.sum(h, axis=0, keepdims=True)
    st1[1:2, :] += jnp.sum(h * h, axis=0, keepdims=True)

  @pl.when(ph == 1)
  def _():
    @pl.when(i == 0)
    def _():
      st2[...] = jnp.zeros_like(st2)
    mean = st1[0:1, :] / n
    var = st1[1:2, :] / n - mean * mean
    h = h1_s[pl.ds(i * _BR, _BR), :]
    h = (h - mean) * lax.rsqrt(var + 1e-5) * g1_ref[...] + be1_ref[...]
    h = jnp.maximum(h, 0.0)
    h2 = jnp.dot(h, w2_ref[...], preferred_element_type=jnp.float32)
    h2 = h2 + b2_ref[...]
    h2_s[pl.ds(i * _BR, _BR), :] = h2
    st2[0:1, :] += jnp.sum(h2, axis=0, keepdims=True)
    st2[1:2, :] += jnp.sum(h2 * h2, axis=0, keepdims=True)

  @pl.when(ph == 2)
  def _():
    mean = st2[0:1, :] / n
    var = st2[1:2, :] / n - mean * mean
    h2 = h2_s[pl.ds(i * _BR, _BR), :]
    out_ref[...] = (h2 - mean) * lax.rsqrt(var + 1e-5) * g2_ref[...] \
        + be2_ref[...]


@jax.jit
def kernel(x, edge_index, epsilon, W1, b1, g1, be1, W2, b2, g2, be2):
  n_nodes, d = x.shape
  n_edges = edge_index.shape[1]
  nw = 32
  per_tile = n_edges // nw
  pad = _NCH_REAL * _CHUNK - per_tile
  filler = _NCH_ARR * _CHUNK - per_tile - pad

  sc_scatter = _make_sc_scatter(n_nodes, d)
  pad_ar = jnp.arange(pad, dtype=jnp.int32)
  src3 = jnp.concatenate([
      edge_index[0].reshape(nw, per_tile),
      jnp.broadcast_to(pad_ar % 128, (nw, pad)),
      jnp.zeros((nw, filler), jnp.int32),
  ], axis=1).reshape(nw, _NCH_ARR, _CHUNK)
  dst3 = jnp.concatenate([
      edge_index[1].reshape(nw, per_tile),
      jnp.broadcast_to(n_nodes + pad_ar % _NTRASH, (nw, pad)),
      jnp.zeros((nw, filler), jnp.int32),
  ], axis=1).reshape(nw, _NCH_ARR, _CHUNK)
  partials = sc_scatter(src3, dst3, x)

  d_hid = W1.shape[1]
  fix = lambda *blk: pl.BlockSpec(blk, lambda ph, i: (0,) * len(blk))
  mlp = pl.pallas_call(
      _mlp_grid_body,
      grid=(3, _NB),
      out_shape=jax.ShapeDtypeStruct((n_nodes, d), jnp.float32),
      in_specs=[
          pl.BlockSpec((1, 1), lambda ph, i: (0, 0),
                       memory_space=pltpu.SMEM),
          pl.BlockSpec((_BR, d),
                       lambda ph, i: (jnp.where(ph == 0, i, 0), 0)),
          pl.BlockSpec((2, _BR, d),
                       lambda ph, i: (0, jnp.where(ph == 0, i, 0), 0)),
          fix(d, d_hid),
          fix(1, d_hid), fix(1, d_hid), fix(1, d_hid),
          fix(d_hid, d),
          fix(1, d), fix(1, d), fix(1, d),
      ],
      out_specs=pl.BlockSpec((_BR, d),
                             lambda ph, i: (jnp.where(ph == 2, i, 0), 0)),
      scratch_shapes=[
          pltpu.VMEM((n_nodes, d_hid), jnp.float32),
          pltpu.VMEM((n_nodes, d), jnp.float32),
          pltpu.VMEM((2, d_hid), jnp.float32),
          pltpu.VMEM((2, d), jnp.float32),
      ],
  )
  return mlp(
      jnp.reshape(epsilon.astype(jnp.float32), (1, 1)),
      x, partials, W1,
      jnp.reshape(b1, (1, d_hid)), jnp.reshape(g1, (1, d_hid)),
      jnp.reshape(be1, (1, d_hid)),
      W2, jnp.reshape(b2, (1, d)), jnp.reshape(g2, (1, d)),
      jnp.reshape(be2, (1, d)))
